# Initial kernel scaffold; baseline (speedup 1.0000x reference)
import jax
import jax.numpy as jnp
from jax.experimental import pallas as pl


def kernel(output, target):
    raise NotImplementedError("write your pallas kernel here")


# speedup vs baseline: 1.2728x; 1.2728x over previous
import functools

import jax
import jax.numpy as jnp
from jax.experimental import pallas as pl
from jax.experimental.pallas import tpu as pltpu

N = 4096
D = 128
ALPHA = 2.0
BETA = 5.0
MARGIN = 0.5
EPSILON = 0.2

BM = 256
NB = N // BM


def _ms_loss_body(x_ref, trow_ref, tcol_ref, out_ref, acc_ref):
    i = pl.program_id(0)

    @pl.when(i == 0)
    def _init():
        acc_ref[0] = 0.0
        acc_ref[1] = 0.0

    x_blk = x_ref[pl.ds(i * BM, BM), :]
    sim = jax.lax.dot_general(
        x_blk, x_ref[...],
        dimension_numbers=(((1,), (1,)), ((), ())),
        preferred_element_type=jnp.float32,
    )

    t_rows = trow_ref[...]
    t_cols = tcol_ref[...]
    same = t_rows == t_cols
    pos_mask = same & (sim < 1.0)
    neg_mask = jnp.logical_not(same)

    inf = jnp.float32(jnp.inf)
    min_pos = jnp.min(jnp.where(pos_mask, sim, inf), axis=1, keepdims=True)
    max_neg = jnp.max(jnp.where(neg_mask, sim, -inf), axis=1, keepdims=True)
    has_pos = jnp.any(pos_mask, axis=1, keepdims=True)
    has_neg = jnp.any(neg_mask, axis=1, keepdims=True)

    neg_sel = neg_mask & ((sim + EPSILON) > min_pos)
    pos_sel = pos_mask & ((sim - EPSILON) < max_neg)
    valid = (has_pos & has_neg
             & jnp.any(neg_sel, axis=1, keepdims=True)
             & jnp.any(pos_sel, axis=1, keepdims=True))

    pos_sum = jnp.sum(
        jnp.where(pos_sel, jnp.exp(-ALPHA * (sim - MARGIN)), 0.0),
        axis=1, keepdims=True)
    neg_sum = jnp.sum(
        jnp.where(neg_sel, jnp.exp(BETA * (sim - MARGIN)), 0.0),
        axis=1, keepdims=True)
    row_loss = jnp.where(
        valid, jnp.log1p(pos_sum) / ALPHA + jnp.log1p(neg_sum) / BETA, 0.0)

    acc_ref[0] += jnp.sum(row_loss)
    acc_ref[1] += jnp.sum(valid.astype(jnp.float32))

    @pl.when(i == NB - 1)
    def _finalize():
        s = acc_ref[0]
        c = acc_ref[1]
        val = jnp.where(c > 0.0, s / jnp.maximum(c, 1.0), 0.0)
        out_ref[...] = jnp.broadcast_to(val, (1, 1))


@functools.partial(jax.jit, static_argnames=("interpret",))
def _ms_loss(x, t, interpret=False):
    trow = t.reshape(N, 1)
    tcol = t.reshape(1, N)
    loss = pl.pallas_call(
        _ms_loss_body,
        grid=(NB,),
        in_specs=[
            pl.BlockSpec((N, D), lambda i: (0, 0)),
            pl.BlockSpec((BM, 1), lambda i: (i, 0)),
            pl.BlockSpec((1, N), lambda i: (0, 0)),
        ],
        out_specs=pl.BlockSpec((1, 1), lambda i: (0, 0)),
        out_shape=jax.ShapeDtypeStruct((1, 1), jnp.float32),
        scratch_shapes=[pltpu.SMEM((2,), jnp.float32)],
        compiler_params=pltpu.CompilerParams(
            dimension_semantics=("arbitrary",)),
        interpret=interpret,
    )(x, trow, tcol)
    return loss[0, 0]


def kernel(output, target):
    return _ms_loss(output, target)


# speedup vs baseline: 1.9956x; 1.5679x over previous
import functools

import jax
import jax.numpy as jnp
from jax.experimental import pallas as pl
from jax.experimental.pallas import tpu as pltpu

N = 4096
D = 128
ALPHA = 2.0
BETA = 5.0
MARGIN = 0.5
EPSILON = 0.2

BM = 256
NB = N // BM


def _ms_loss_body(x_ref, trow_ref, tcol_ref, out_ref, acc_ref):
    i = pl.program_id(0)

    @pl.when(i == 0)
    def _init():
        acc_ref[0] = 0.0
        acc_ref[1] = 0.0

    x_blk = x_ref[pl.ds(i * BM, BM), :]
    sim = jax.lax.dot_general(
        x_blk, x_ref[...],
        dimension_numbers=(((1,), (1,)), ((), ())),
        preferred_element_type=jnp.float32,
    )

    t_rows = trow_ref[...]
    t_cols = tcol_ref[...]
    same = t_rows == t_cols

    inf = jnp.float32(jnp.inf)
    min_pos = jnp.min(
        jnp.where(same & (sim < 1.0), sim, inf), axis=1, keepdims=True)
    max_neg = jnp.max(jnp.where(same, -inf, sim), axis=1, keepdims=True)

    thr_p = jnp.minimum(jnp.float32(1.0), max_neg + EPSILON)
    thr_n = min_pos - EPSILON

    log2e = 1.4426950408889634
    pexp = jnp.exp2(sim * jnp.float32(-ALPHA * log2e)
                    + jnp.float32(ALPHA * MARGIN * log2e))
    nexp = jnp.exp2(sim * jnp.float32(BETA * log2e)
                    + jnp.float32(-BETA * MARGIN * log2e))

    pos_sum = jnp.sum(
        jnp.where(same & (sim < thr_p), pexp, 0.0), axis=1, keepdims=True)
    neg_sum = jnp.sum(
        jnp.where(same, 0.0, jnp.where(sim > thr_n, nexp, 0.0)),
        axis=1, keepdims=True)

    valid = ((min_pos < inf) & (max_neg > -inf)
             & (pos_sum > 0.0) & (neg_sum > 0.0))
    row_loss = jnp.where(
        valid,
        jnp.log1p(pos_sum) * jnp.float32(1.0 / ALPHA)
        + jnp.log1p(neg_sum) * jnp.float32(1.0 / BETA),
        0.0)

    acc_ref[0] += jnp.sum(row_loss)
    acc_ref[1] += jnp.sum(valid.astype(jnp.float32))

    @pl.when(i == NB - 1)
    def _finalize():
        s = acc_ref[0]
        c = acc_ref[1]
        val = jnp.where(c > 0.0, s / jnp.maximum(c, 1.0), 0.0)
        out_ref[...] = jnp.broadcast_to(val, (1, 1))


@functools.partial(jax.jit, static_argnames=("interpret",))
def _ms_loss(x, t, interpret=False):
    trow = t.reshape(N, 1)
    tcol = t.reshape(1, N)
    loss = pl.pallas_call(
        _ms_loss_body,
        grid=(NB,),
        in_specs=[
            pl.BlockSpec((N, D), lambda i: (0, 0)),
            pl.BlockSpec((BM, 1), lambda i: (i, 0)),
            pl.BlockSpec((1, N), lambda i: (0, 0)),
        ],
        out_specs=pl.BlockSpec((1, 1), lambda i: (0, 0)),
        out_shape=jax.ShapeDtypeStruct((1, 1), jnp.float32),
        scratch_shapes=[pltpu.SMEM((2,), jnp.float32)],
        compiler_params=pltpu.CompilerParams(
            dimension_semantics=("arbitrary",)),
        interpret=interpret,
    )(x, trow, tcol)
    return loss[0, 0]


def kernel(output, target):
    return _ms_loss(output, target)


# speedup vs baseline: 2.2402x; 1.1225x over previous
import functools

import jax
import jax.numpy as jnp
from jax.experimental import pallas as pl
from jax.experimental.pallas import tpu as pltpu

N = 4096
D = 128
ALPHA = 2.0
BETA = 5.0
MARGIN = 0.5
EPSILON = 0.2

BM = 512
NB = N // BM


def _ms_loss_body(x_ref, trow_ref, tcol_ref, out_ref, acc_ref):
    i = pl.program_id(0)

    @pl.when(i == 0)
    def _init():
        acc_ref[0] = 0.0
        acc_ref[1] = 0.0

    x_blk = x_ref[pl.ds(i * BM, BM), :]
    sim = jax.lax.dot_general(
        x_blk, x_ref[...],
        dimension_numbers=(((1,), (1,)), ((), ())),
        preferred_element_type=jnp.float32,
    )

    t_rows = trow_ref[...]
    t_cols = tcol_ref[...]
    same = t_rows == t_cols

    inf = jnp.float32(jnp.inf)
    min_pos = jnp.min(
        jnp.where(same & (sim < 1.0), sim, inf), axis=1, keepdims=True)
    max_neg = jnp.max(jnp.where(same, -inf, sim), axis=1, keepdims=True)

    thr_p = jnp.minimum(jnp.float32(1.0), max_neg + EPSILON)
    thr_n = min_pos - EPSILON

    log2e = 1.4426950408889634
    pexp = jnp.exp2(sim * jnp.float32(-ALPHA * log2e)
                    + jnp.float32(ALPHA * MARGIN * log2e))
    nexp = jnp.exp2(sim * jnp.float32(BETA * log2e)
                    + jnp.float32(-BETA * MARGIN * log2e))

    pos_sum = jnp.sum(
        jnp.where(same & (sim < thr_p), pexp, 0.0), axis=1, keepdims=True)
    neg_sum = jnp.sum(
        jnp.where(same, 0.0, jnp.where(sim > thr_n, nexp, 0.0)),
        axis=1, keepdims=True)

    valid = ((min_pos < inf) & (max_neg > -inf)
             & (pos_sum > 0.0) & (neg_sum > 0.0))
    row_loss = jnp.where(
        valid,
        jnp.log1p(pos_sum) * jnp.float32(1.0 / ALPHA)
        + jnp.log1p(neg_sum) * jnp.float32(1.0 / BETA),
        0.0)

    acc_ref[0] += jnp.sum(row_loss)
    acc_ref[1] += jnp.sum(valid.astype(jnp.float32))

    @pl.when(i == NB - 1)
    def _finalize():
        s = acc_ref[0]
        c = acc_ref[1]
        val = jnp.where(c > 0.0, s / jnp.maximum(c, 1.0), 0.0)
        out_ref[...] = jnp.broadcast_to(val, (1, 1))


@functools.partial(jax.jit, static_argnames=("interpret",))
def _ms_loss(x, t, interpret=False):
    trow = t.reshape(N, 1)
    tcol = t.reshape(1, N)
    loss = pl.pallas_call(
        _ms_loss_body,
        grid=(NB,),
        in_specs=[
            pl.BlockSpec((N, D), lambda i: (0, 0)),
            pl.BlockSpec((BM, 1), lambda i: (i, 0)),
            pl.BlockSpec((1, N), lambda i: (0, 0)),
        ],
        out_specs=pl.BlockSpec((1, 1), lambda i: (0, 0)),
        out_shape=jax.ShapeDtypeStruct((1, 1), jnp.float32),
        scratch_shapes=[pltpu.SMEM((2,), jnp.float32)],
        compiler_params=pltpu.CompilerParams(
            dimension_semantics=("arbitrary",)),
        interpret=interpret,
    )(x, trow, tcol)
    return loss[0, 0]


def kernel(output, target):
    return _ms_loss(output, target)


# speedup vs baseline: 2.4109x; 1.0762x over previous
import functools

import jax
import jax.numpy as jnp
from jax.experimental import pallas as pl
from jax.experimental.pallas import tpu as pltpu

N = 4096
D = 128
ALPHA = 2.0
BETA = 5.0
MARGIN = 0.5
EPSILON = 0.2

BM = 1024
NB = N // BM


def _ms_loss_body(x_ref, trow_ref, tcol_ref, out_ref, acc_ref):
    i = pl.program_id(0)

    @pl.when(i == 0)
    def _init():
        acc_ref[0] = 0.0
        acc_ref[1] = 0.0

    x_blk = x_ref[pl.ds(i * BM, BM), :]
    sim = jax.lax.dot_general(
        x_blk, x_ref[...],
        dimension_numbers=(((1,), (1,)), ((), ())),
        preferred_element_type=jnp.float32,
    )

    t_rows = trow_ref[...]
    t_cols = tcol_ref[...]
    same = t_rows == t_cols

    inf = jnp.float32(jnp.inf)
    min_pos = jnp.min(
        jnp.where(same & (sim < 1.0), sim, inf), axis=1, keepdims=True)
    max_neg = jnp.max(jnp.where(same, -inf, sim), axis=1, keepdims=True)

    thr_p = jnp.minimum(jnp.float32(1.0), max_neg + EPSILON)
    thr_n = min_pos - EPSILON

    log2e = 1.4426950408889634
    pexp = jnp.exp2(sim * jnp.float32(-ALPHA * log2e)
                    + jnp.float32(ALPHA * MARGIN * log2e))
    nexp = jnp.exp2(sim * jnp.float32(BETA * log2e)
                    + jnp.float32(-BETA * MARGIN * log2e))

    pos_sum = jnp.sum(
        jnp.where(same & (sim < thr_p), pexp, 0.0), axis=1, keepdims=True)
    neg_sum = jnp.sum(
        jnp.where(same, 0.0, jnp.where(sim > thr_n, nexp, 0.0)),
        axis=1, keepdims=True)

    valid = ((min_pos < inf) & (max_neg > -inf)
             & (pos_sum > 0.0) & (neg_sum > 0.0))
    row_loss = jnp.where(
        valid,
        jnp.log1p(pos_sum) * jnp.float32(1.0 / ALPHA)
        + jnp.log1p(neg_sum) * jnp.float32(1.0 / BETA),
        0.0)

    acc_ref[0] += jnp.sum(row_loss)
    acc_ref[1] += jnp.sum(valid.astype(jnp.float32))

    @pl.when(i == NB - 1)
    def _finalize():
        s = acc_ref[0]
        c = acc_ref[1]
        val = jnp.where(c > 0.0, s / jnp.maximum(c, 1.0), 0.0)
        out_ref[...] = jnp.broadcast_to(val, (1, 1))


@functools.partial(jax.jit, static_argnames=("interpret",))
def _ms_loss(x, t, interpret=False):
    trow = t.reshape(N, 1)
    tcol = t.reshape(1, N)
    loss = pl.pallas_call(
        _ms_loss_body,
        grid=(NB,),
        in_specs=[
            pl.BlockSpec((N, D), lambda i: (0, 0)),
            pl.BlockSpec((BM, 1), lambda i: (i, 0)),
            pl.BlockSpec((1, N), lambda i: (0, 0)),
        ],
        out_specs=pl.BlockSpec((1, 1), lambda i: (0, 0)),
        out_shape=jax.ShapeDtypeStruct((1, 1), jnp.float32),
        scratch_shapes=[pltpu.SMEM((2,), jnp.float32)],
        compiler_params=pltpu.CompilerParams(
            dimension_semantics=("arbitrary",)),
        interpret=interpret,
    )(x, trow, tcol)
    return loss[0, 0]


def kernel(output, target):
    return _ms_loss(output, target)
